# baseline (speedup 1.0000x reference)
import functools

import jax
import jax.numpy as jnp
from jax import lax
from jax.experimental import pallas as pl
from jax.experimental.pallas import tpu as pltpu
from jax.experimental.pallas import tpu_sc as plsc

_V = 1000000
_D = 64
_B = 4096
_S = 200
_NW = 32

_FULL_CHUNKS = _V // 128
_TAIL = _V - _FULL_CHUNKS * 128
_PER_W = 244
_EXTRA = _FULL_CHUNKS - _PER_W * _NW


def _repack_table():
    mesh = plsc.VectorSubcoreMesh(core_axis_name="c", subcore_axis_name="s")

    @functools.partial(
        pl.kernel,
        mesh=mesh,
        out_type=jax.ShapeDtypeStruct((_V * _D,), jnp.float32),
        scratch_types=[
            [pltpu.VMEM((_D, 128), jnp.float32) for _ in range(2)],
            [pltpu.VMEM((8192,), jnp.float32) for _ in range(2)],
            [pltpu.SemaphoreType.DMA for _ in range(2)],
            [pltpu.SemaphoreType.DMA for _ in range(2)],
        ],
        compiler_params=pltpu.CompilerParams(
            use_tc_tiling_on_sc=True,
            needs_layout_passes=False,
            disable_bounds_checks=True,
        ),
    )
    def k(wt_hbm, wtail_hbm, scr_hbm, ibufs, obufs, isems, osems):
        nc = lax.axis_size("c")
        wid = lax.axis_index("s") * nc + lax.axis_index("c")
        lo = wid * _PER_W

        iota = lax.iota(jnp.int32, 16)

        def fire_in(c, b):
            pltpu.async_copy(
                wt_hbm.at[:, pl.ds(c * 128, 128)], ibufs[b], isems[b]
            )

        def wait_in(c, b):
            pltpu.make_async_copy(
                wt_hbm.at[:, pl.ds(c * 128, 128)], ibufs[b], isems[b]
            ).wait()

        def transpose_chunk(b, n_h):
            @pl.loop(0, n_h)
            def _(h):
                for d in range(_D):
                    x = ibufs[b][d, pl.ds(16 * h, 16)]
                    plsc.store_scatter(
                        obufs[b], [1024 * h + 64 * iota + d], x
                    )

        def fire_out(c, b, words):
            pltpu.async_copy(
                obufs[b].at[pl.ds(0, words)],
                scr_hbm.at[pl.ds(c * 8192, words)],
                osems[b],
            )

        def wait_out(c, b, words):
            pltpu.make_async_copy(
                obufs[b].at[pl.ds(0, words)],
                scr_hbm.at[pl.ds(c * 8192, words)],
                osems[b],
            ).wait()

        fire_in(lo, 0)
        fire_in(lo + 1, 1)

        @pl.loop(0, _PER_W, step=2)
        def _(i):
            for b in range(2):
                j = i + b
                c = lo + j
                wait_in(c, b)

                @pl.when(j >= 2)
                def _():
                    wait_out(c - 2, b, 8192)

                transpose_chunk(b, 8)
                fire_out(c, b, 8192)

                @pl.when(j + 2 < _PER_W)
                def _():
                    fire_in(c + 2, b)

        wait_out(lo + _PER_W - 2, 0, 8192)
        wait_out(lo + _PER_W - 1, 1, 8192)

        for w in range(_EXTRA):
            @pl.when(wid == w)
            def _(w=w):
                c = _NW * _PER_W + w
                pltpu.sync_copy(wt_hbm.at[:, pl.ds(c * 128, 128)], ibufs[0])
                transpose_chunk(0, 8)
                fire_out(c, 0, 8192)
                wait_out(c, 0, 8192)

        @pl.when(wid == _NW - 1)
        def _():
            pltpu.sync_copy(wtail_hbm, ibufs[0])
            transpose_chunk(0, _TAIL // 16)
            fire_out(_FULL_CHUNKS, 0, _TAIL * _D)
            wait_out(_FULL_CHUNKS, 0, _TAIL * _D)

    return k


def _gather_native():
    mesh = plsc.VectorSubcoreMesh(core_axis_name="c", subcore_axis_name="s")

    @functools.partial(
        pl.kernel,
        mesh=mesh,
        out_type=jax.ShapeDtypeStruct((_S, 8, _NW, 8, 128), jnp.float32),
        scratch_types=[
            [pltpu.VMEM((128,), jnp.int32) for _ in range(2)],
            [pltpu.VMEM((128, _D), jnp.float32) for _ in range(2)],
            [pltpu.VMEM((8, 8, 128), jnp.float32) for _ in range(2)],
            [pltpu.SemaphoreType.DMA for _ in range(2)],
            [pltpu.SemaphoreType.DMA for _ in range(2)],
        ],
        compiler_params=pltpu.CompilerParams(use_tc_tiling_on_sc=False, needs_layout_passes=False),
    )
    def k(idx_hbm, tab_hbm, out_hbm, idxbufs, rows, obufs, gsems, osems):
        nc = lax.axis_size("c")
        wid = lax.axis_index("s") * nc + lax.axis_index("c")

        iota = lax.iota(jnp.int32, 16)

        def fire_gather(s, b):
            pltpu.sync_copy(idx_hbm.at[s, pl.ds(wid * 128, 128)], idxbufs[b])
            pltpu.async_copy(tab_hbm.at[idxbufs[b]], rows[b], gsems[b])

        def wait_gather(b):
            pltpu.make_async_copy(
                tab_hbm.at[idxbufs[b]], rows[b], gsems[b]
            ).wait()

        def fire_out(s, b):
            pltpu.async_copy(obufs[b], out_hbm.at[s, :, wid], osems[b])

        def wait_out(s, b):
            pltpu.make_async_copy(
                obufs[b], out_hbm.at[s, :, wid], osems[b]
            ).wait()

        def transpose_task(b):
            @pl.loop(0, 8)
            def _(h):
                for d in range(_D):
                    x = plsc.load_gather(
                        rows[b], [16 * h + iota, jnp.full((16,), d, jnp.int32)]
                    )
                    obufs[b][d // 8, d % 8, pl.ds(16 * h, 16)] = x

        fire_gather(0, 0)
        fire_gather(1, 1)

        @pl.loop(0, _S, step=2)
        def _(i):
            for b in range(2):
                s = i + b
                wait_gather(b)

                @pl.when(s >= 2)
                def _():
                    wait_out(s - 2, b)

                transpose_task(b)
                fire_out(s, b)

                @pl.when(s + 2 < _S)
                def _():
                    fire_gather(s + 2, b)

        wait_out(_S - 2, 0)
        wait_out(_S - 1, 1)

    return k


def kernel(token_ids, weights):
    wt = weights.T
    wtail = jnp.pad(weights[_V - _TAIL:], ((0, 128 - _TAIL), (0, 0))).T
    scr = _repack_table()(wt, wtail)
    tab = scr.reshape(_V, _D)
    idx_t = token_ids.T.astype(jnp.int32)
    out5 = _gather_native()(idx_t, tab)
    return out5.transpose(2, 4, 0, 1, 3).reshape(_B, _S, _D)


# speedup vs baseline: 1.5535x; 1.5535x over previous
import functools

import jax
import jax.numpy as jnp
from jax import lax
from jax.experimental import pallas as pl
from jax.experimental.pallas import tpu as pltpu
from jax.experimental.pallas import tpu_sc as plsc

_V = 1000000
_D = 64
_B = 4096
_S = 200
_NW = 32

_FULL_CHUNKS = _V // 128
_TAIL = _V - _FULL_CHUNKS * 128
_PER_W = 244
_EXTRA = _FULL_CHUNKS - _PER_W * _NW


def _repack_table():
    mesh = plsc.VectorSubcoreMesh(core_axis_name="c", subcore_axis_name="s")

    @functools.partial(
        pl.kernel,
        mesh=mesh,
        out_type=jax.ShapeDtypeStruct((_V * _D,), jnp.float32),
        scratch_types=[
            [pltpu.VMEM((_D, 128), jnp.float32) for _ in range(2)],
            [pltpu.VMEM((8192,), jnp.float32) for _ in range(2)],
            [pltpu.SemaphoreType.DMA for _ in range(2)],
            [pltpu.SemaphoreType.DMA for _ in range(2)],
        ],
        compiler_params=pltpu.CompilerParams(
            use_tc_tiling_on_sc=True,
            needs_layout_passes=False,
            disable_bounds_checks=True,
        ),
    )
    def k(wt_hbm, wtail_hbm, scr_hbm, ibufs, obufs, isems, osems):
        nc = lax.axis_size("c")
        wid = lax.axis_index("s") * nc + lax.axis_index("c")
        lo = wid * _PER_W

        iota = lax.iota(jnp.int32, 16)

        def fire_in(c, b):
            pltpu.async_copy(
                wt_hbm.at[:, pl.ds(c * 128, 128)], ibufs[b], isems[b]
            )

        def wait_in(c, b):
            pltpu.make_async_copy(
                wt_hbm.at[:, pl.ds(c * 128, 128)], ibufs[b], isems[b]
            ).wait()

        def transpose_chunk(b, n_h):
            @pl.loop(0, n_h)
            def _(h):
                @plsc.parallel_loop(0, _D, unroll=8)
                def _(d):
                    x = ibufs[b][d, pl.ds(16 * h, 16)]
                    plsc.store_scatter(
                        obufs[b], [1024 * h + 64 * iota + d], x
                    )

        def fire_out(c, b, words):
            pltpu.async_copy(
                obufs[b].at[pl.ds(0, words)],
                scr_hbm.at[pl.ds(c * 8192, words)],
                osems[b],
            )

        def wait_out(c, b, words):
            pltpu.make_async_copy(
                obufs[b].at[pl.ds(0, words)],
                scr_hbm.at[pl.ds(c * 8192, words)],
                osems[b],
            ).wait()

        fire_in(lo, 0)
        fire_in(lo + 1, 1)

        @pl.loop(0, _PER_W, step=2)
        def _(i):
            for b in range(2):
                j = i + b
                c = lo + j
                wait_in(c, b)

                @pl.when(j >= 2)
                def _():
                    wait_out(c - 2, b, 8192)

                transpose_chunk(b, 8)
                fire_out(c, b, 8192)

                @pl.when(j + 2 < _PER_W)
                def _():
                    fire_in(c + 2, b)

        wait_out(lo + _PER_W - 2, 0, 8192)
        wait_out(lo + _PER_W - 1, 1, 8192)

        for w in range(_EXTRA):
            @pl.when(wid == w)
            def _(w=w):
                c = _NW * _PER_W + w
                pltpu.sync_copy(wt_hbm.at[:, pl.ds(c * 128, 128)], ibufs[0])
                transpose_chunk(0, 8)
                fire_out(c, 0, 8192)
                wait_out(c, 0, 8192)

        @pl.when(wid == _NW - 1)
        def _():
            pltpu.sync_copy(wtail_hbm, ibufs[0])
            transpose_chunk(0, _TAIL // 16)
            fire_out(_FULL_CHUNKS, 0, _TAIL * _D)
            wait_out(_FULL_CHUNKS, 0, _TAIL * _D)

    return k


def _gather_native():
    mesh = plsc.VectorSubcoreMesh(core_axis_name="c", subcore_axis_name="s")

    @functools.partial(
        pl.kernel,
        mesh=mesh,
        out_type=jax.ShapeDtypeStruct((_S, 8, _NW, 8, 128), jnp.float32),
        scratch_types=[
            [pltpu.VMEM((128,), jnp.int32) for _ in range(2)],
            [pltpu.VMEM((128, _D), jnp.float32) for _ in range(2)],
            [pltpu.VMEM((8, 8, 128), jnp.float32) for _ in range(2)],
            [pltpu.SemaphoreType.DMA for _ in range(2)],
            [pltpu.SemaphoreType.DMA for _ in range(2)],
        ],
        compiler_params=pltpu.CompilerParams(use_tc_tiling_on_sc=False, needs_layout_passes=False),
    )
    def k(idx_hbm, tab_hbm, out_hbm, idxbufs, rows, obufs, gsems, osems):
        nc = lax.axis_size("c")
        wid = lax.axis_index("s") * nc + lax.axis_index("c")

        iota = lax.iota(jnp.int32, 16)

        def fire_gather(s, b):
            pltpu.sync_copy(idx_hbm.at[s, pl.ds(wid * 128, 128)], idxbufs[b])
            pltpu.async_copy(tab_hbm.at[idxbufs[b]], rows[b], gsems[b])

        def wait_gather(b):
            pltpu.make_async_copy(
                tab_hbm.at[idxbufs[b]], rows[b], gsems[b]
            ).wait()

        def fire_out(s, b):
            pltpu.async_copy(obufs[b], out_hbm.at[s, :, wid], osems[b])

        def wait_out(s, b):
            pltpu.make_async_copy(
                obufs[b], out_hbm.at[s, :, wid], osems[b]
            ).wait()

        def transpose_task(b):
            @pl.loop(0, 8)
            def _(h):
                @plsc.parallel_loop(0, _D, unroll=8)
                def _(d):
                    x = plsc.load_gather(
                        rows[b],
                        [16 * h + iota, d + jnp.zeros((16,), jnp.int32)],
                    )
                    obufs[b][lax.div(d, 8), lax.rem(d, 8), pl.ds(16 * h, 16)] = x

        fire_gather(0, 0)
        fire_gather(1, 1)

        @pl.loop(0, _S, step=2)
        def _(i):
            for b in range(2):
                s = i + b
                wait_gather(b)

                @pl.when(s >= 2)
                def _():
                    wait_out(s - 2, b)

                transpose_task(b)
                fire_out(s, b)

                @pl.when(s + 2 < _S)
                def _():
                    fire_gather(s + 2, b)

        wait_out(_S - 2, 0)
        wait_out(_S - 1, 1)

    return k


def kernel(token_ids, weights):
    wt = weights.T
    wtail = jnp.pad(weights[_V - _TAIL:], ((0, 128 - _TAIL), (0, 0))).T
    scr = _repack_table()(wt, wtail)
    tab = scr.reshape(_V, _D)
    idx_t = token_ids.T.astype(jnp.int32)
    out5 = _gather_native()(idx_t, tab)
    return out5.transpose(2, 4, 0, 1, 3).reshape(_B, _S, _D)


# speedup vs baseline: 1.5722x; 1.0121x over previous
import functools

import jax
import jax.numpy as jnp
from jax import lax
from jax.experimental import pallas as pl
from jax.experimental.pallas import tpu as pltpu
from jax.experimental.pallas import tpu_sc as plsc

_V = 1000000
_D = 64
_B = 4096
_S = 200
_NW = 32

_FULL_CHUNKS = _V // 128
_TAIL = _V - _FULL_CHUNKS * 128
_PER_W = 244
_EXTRA = _FULL_CHUNKS - _PER_W * _NW


def _repack_table():
    mesh = plsc.VectorSubcoreMesh(core_axis_name="c", subcore_axis_name="s")

    @functools.partial(
        pl.kernel,
        mesh=mesh,
        out_type=jax.ShapeDtypeStruct((_V * _D,), jnp.float32),
        scratch_types=[
            [pltpu.VMEM((_D, 128), jnp.float32) for _ in range(2)],
            [pltpu.VMEM((8192,), jnp.float32) for _ in range(2)],
            [pltpu.SemaphoreType.DMA for _ in range(2)],
            [pltpu.SemaphoreType.DMA for _ in range(2)],
        ],
        compiler_params=pltpu.CompilerParams(
            use_tc_tiling_on_sc=True,
            needs_layout_passes=False,
            disable_bounds_checks=True,
        ),
    )
    def k(wt_hbm, wtail_hbm, scr_hbm, ibufs, obufs, isems, osems):
        nc = lax.axis_size("c")
        wid = lax.axis_index("s") * nc + lax.axis_index("c")
        lo = wid * _PER_W

        iota = lax.iota(jnp.int32, 16)

        def fire_in(c, b):
            pltpu.async_copy(
                wt_hbm.at[:, pl.ds(c * 128, 128)], ibufs[b], isems[b]
            )

        def wait_in(c, b):
            pltpu.make_async_copy(
                wt_hbm.at[:, pl.ds(c * 128, 128)], ibufs[b], isems[b]
            ).wait()

        def transpose_chunk(b, n_h):
            @plsc.parallel_loop(0, _D, unroll=4)
            def _(d):
                base = 64 * iota + d
                for h in range(n_h):
                    x = ibufs[b][d, pl.ds(16 * h, 16)]
                    plsc.store_scatter(obufs[b], [base + 1024 * h], x)

        def fire_out(c, b, words):
            pltpu.async_copy(
                obufs[b].at[pl.ds(0, words)],
                scr_hbm.at[pl.ds(c * 8192, words)],
                osems[b],
            )

        def wait_out(c, b, words):
            pltpu.make_async_copy(
                obufs[b].at[pl.ds(0, words)],
                scr_hbm.at[pl.ds(c * 8192, words)],
                osems[b],
            ).wait()

        fire_in(lo, 0)
        fire_in(lo + 1, 1)

        @pl.loop(0, _PER_W, step=2)
        def _(i):
            for b in range(2):
                j = i + b
                c = lo + j
                wait_in(c, b)

                @pl.when(j >= 2)
                def _():
                    wait_out(c - 2, b, 8192)

                transpose_chunk(b, 8)
                fire_out(c, b, 8192)

                @pl.when(j + 2 < _PER_W)
                def _():
                    fire_in(c + 2, b)

        wait_out(lo + _PER_W - 2, 0, 8192)
        wait_out(lo + _PER_W - 1, 1, 8192)

        for w in range(_EXTRA):
            @pl.when(wid == w)
            def _(w=w):
                c = _NW * _PER_W + w
                pltpu.sync_copy(wt_hbm.at[:, pl.ds(c * 128, 128)], ibufs[0])
                transpose_chunk(0, 8)
                fire_out(c, 0, 8192)
                wait_out(c, 0, 8192)

        @pl.when(wid == _NW - 1)
        def _():
            pltpu.sync_copy(wtail_hbm, ibufs[0])
            transpose_chunk(0, _TAIL // 16)
            fire_out(_FULL_CHUNKS, 0, _TAIL * _D)
            wait_out(_FULL_CHUNKS, 0, _TAIL * _D)

    return k


def _gather_native():
    mesh = plsc.VectorSubcoreMesh(core_axis_name="c", subcore_axis_name="s")

    @functools.partial(
        pl.kernel,
        mesh=mesh,
        out_type=jax.ShapeDtypeStruct((_S, 8, _NW, 8, 128), jnp.float32),
        scratch_types=[
            [pltpu.VMEM((128,), jnp.int32) for _ in range(2)],
            [pltpu.VMEM((128, _D), jnp.float32) for _ in range(2)],
            [pltpu.VMEM((8, 8, 128), jnp.float32) for _ in range(2)],
            [pltpu.SemaphoreType.DMA for _ in range(2)],
            [pltpu.SemaphoreType.DMA for _ in range(2)],
        ],
        compiler_params=pltpu.CompilerParams(use_tc_tiling_on_sc=False, needs_layout_passes=False),
    )
    def k(idx_hbm, tab_hbm, out_hbm, idxbufs, rows, obufs, gsems, osems):
        nc = lax.axis_size("c")
        wid = lax.axis_index("s") * nc + lax.axis_index("c")

        iota = lax.iota(jnp.int32, 16)

        def fire_gather(s, b):
            pltpu.sync_copy(idx_hbm.at[s, pl.ds(wid * 128, 128)], idxbufs[b])
            pltpu.async_copy(tab_hbm.at[idxbufs[b]], rows[b], gsems[b])

        def wait_gather(b):
            pltpu.make_async_copy(
                tab_hbm.at[idxbufs[b]], rows[b], gsems[b]
            ).wait()

        def fire_out(s, b):
            pltpu.async_copy(obufs[b], out_hbm.at[s, :, wid], osems[b])

        def wait_out(s, b):
            pltpu.make_async_copy(
                obufs[b], out_hbm.at[s, :, wid], osems[b]
            ).wait()

        def transpose_task(b):
            @plsc.parallel_loop(0, _D, unroll=4)
            def _(d):
                dvec = d + jnp.zeros((16,), jnp.int32)
                di = lax.div(d, 8)
                dj = lax.rem(d, 8)
                for h in range(8):
                    x = plsc.load_gather(rows[b], [16 * h + iota, dvec])
                    obufs[b][di, dj, pl.ds(16 * h, 16)] = x

        fire_gather(0, 0)
        fire_gather(1, 1)

        @pl.loop(0, _S, step=2)
        def _(i):
            for b in range(2):
                s = i + b
                wait_gather(b)

                @pl.when(s >= 2)
                def _():
                    wait_out(s - 2, b)

                transpose_task(b)
                fire_out(s, b)

                @pl.when(s + 2 < _S)
                def _():
                    fire_gather(s + 2, b)

        wait_out(_S - 2, 0)
        wait_out(_S - 1, 1)

    return k


def kernel(token_ids, weights):
    wt = weights.T
    wtail = jnp.pad(weights[_V - _TAIL:], ((0, 128 - _TAIL), (0, 0))).T
    scr = _repack_table()(wt, wtail)
    tab = scr.reshape(_V, _D)
    idx_t = token_ids.T.astype(jnp.int32)
    out5 = _gather_native()(idx_t, tab)
    return out5.transpose(2, 4, 0, 1, 3).reshape(_B, _S, _D)


# speedup vs baseline: 2.0365x; 1.2953x over previous
import functools

import jax
import jax.numpy as jnp
from jax import lax
from jax.experimental import pallas as pl
from jax.experimental.pallas import tpu as pltpu
from jax.experimental.pallas import tpu_sc as plsc

_NBUF = 2


def _gather_kernel(N, D, n_per_w, chunk, n_chunks):
    mesh = plsc.VectorSubcoreMesh(core_axis_name="c", subcore_axis_name="s")

    @functools.partial(
        pl.kernel,
        mesh=mesh,
        out_type=jax.ShapeDtypeStruct((N, D), jnp.float32),
        scratch_types=[
            [pltpu.VMEM((chunk,), jnp.int32) for _ in range(_NBUF)],
            [pltpu.VMEM((chunk, D), jnp.float32) for _ in range(_NBUF)],
            [pltpu.SemaphoreType.DMA for _ in range(_NBUF)],
        ],
        compiler_params=pltpu.CompilerParams(use_tc_tiling_on_sc=False, needs_layout_passes=False),
    )
    def k(idx_hbm, tab_hbm, out_hbm, idx_bufs, rows, sems):
        nc = lax.axis_size("c")
        wid = lax.axis_index("s") * nc + lax.axis_index("c")
        base = wid * n_per_w

        def fire(j, b):
            pltpu.sync_copy(idx_hbm.at[pl.ds(base + j * chunk, chunk)], idx_bufs[b])
            pltpu.async_copy(tab_hbm.at[idx_bufs[b]], rows[b], sems[b])

        for b in range(_NBUF):
            fire(b, b)

        @pl.loop(0, n_chunks, step=_NBUF)
        def _(i):
            for b in range(_NBUF):
                j = i + b
                pltpu.make_async_copy(
                    tab_hbm.at[idx_bufs[b]], rows[b], sems[b]
                ).wait()
                pltpu.sync_copy(rows[b], out_hbm.at[pl.ds(base + j * chunk, chunk)])

                @pl.when(j + _NBUF < n_chunks)
                def _():
                    fire(j + _NBUF, b)

    return k


def kernel(token_ids, weights):
    B, S = token_ids.shape
    V, D = weights.shape
    N = B * S
    idx = token_ids.reshape(N).astype(jnp.int32)

    NW = 32
    n_per_w = N // NW
    chunk = 512
    n_chunks = n_per_w // chunk
    assert n_per_w % chunk == 0 and n_chunks % _NBUF == 0

    out = _gather_kernel(N, D, n_per_w, chunk, n_chunks)(idx, weights)
    return out.reshape(B, S, D)


# speedup vs baseline: 2.0452x; 1.0043x over previous
import functools

import jax
import jax.numpy as jnp
from jax import lax
from jax.experimental import pallas as pl
from jax.experimental.pallas import tpu as pltpu
from jax.experimental.pallas import tpu_sc as plsc

_NBUF = 2


def _gather_kernel(N, D, n_per_w, chunk, n_chunks):
    mesh = plsc.VectorSubcoreMesh(core_axis_name="c", subcore_axis_name="s")

    @functools.partial(
        pl.kernel,
        mesh=mesh,
        out_type=jax.ShapeDtypeStruct((N, D), jnp.float32),
        scratch_types=[
            [pltpu.VMEM((chunk,), jnp.int32) for _ in range(_NBUF)],
            [pltpu.VMEM((chunk, D), jnp.float32) for _ in range(_NBUF)],
            [pltpu.SemaphoreType.DMA for _ in range(_NBUF)],
        ],
        compiler_params=pltpu.CompilerParams(use_tc_tiling_on_sc=False, needs_layout_passes=False),
    )
    def k(idx_hbm, tab_hbm, out_hbm, idx_bufs, rows, sems):
        nc = lax.axis_size("c")
        wid = lax.axis_index("s") * nc + lax.axis_index("c")
        base = wid * n_per_w

        def fire(j, b):
            pltpu.sync_copy(idx_hbm.at[pl.ds(base + j * chunk, chunk)], idx_bufs[b])
            pltpu.async_copy(tab_hbm.at[idx_bufs[b]], rows[b], sems[b])

        for b in range(_NBUF):
            fire(b, b)

        @pl.loop(0, n_chunks, step=_NBUF)
        def _(i):
            for b in range(_NBUF):
                j = i + b
                pltpu.make_async_copy(
                    tab_hbm.at[idx_bufs[b]], rows[b], sems[b]
                ).wait()
                pltpu.sync_copy(rows[b], out_hbm.at[pl.ds(base + j * chunk, chunk)])

                @pl.when(j + _NBUF < n_chunks)
                def _():
                    fire(j + _NBUF, b)

    return k


def kernel(token_ids, weights):
    B, S = token_ids.shape
    V, D = weights.shape
    N = B * S
    idx = token_ids.reshape(N).astype(jnp.int32)

    NW = 32
    n_per_w = N // NW
    chunk = 800
    n_chunks = n_per_w // chunk
    assert n_per_w % chunk == 0 and n_chunks % _NBUF == 0

    out = _gather_kernel(N, D, n_per_w, chunk, n_chunks)(idx, weights)
    return out.reshape(B, S, D)
